# Initial kernel scaffold; baseline (speedup 1.0000x reference)
import jax
import jax.numpy as jnp
from jax.experimental import pallas as pl


def kernel(x, edge_index, W1_lin, b1_lin, W1_mod, b1_mod, W2_lin, b2_lin, W2_mod, b2_mod, W3_lin, b3_lin, W3_mod, b3_mod):
    raise NotImplementedError("write your pallas kernel here")


# speedup vs baseline: 17.3898x; 17.3898x over previous
import functools

import jax
import jax.numpy as jnp
from jax import lax
from jax.experimental import pallas as pl
from jax.experimental.pallas import tpu as pltpu
from jax.experimental.pallas import tpu_sc as plsc

NC = 2
NS = 16
CH = 128
GS = 8


def _round_up(a, m):
    return (a + m - 1) // m * m


def _make_agg16(rows, n_chunk_rows, chunks_per_tile):
    rpt = rows // NS
    zb = rpt // 8
    n_groups = chunks_per_tile // GS

    mesh = plsc.VectorSubcoreMesh(core_axis_name="c", subcore_axis_name="s")

    @functools.partial(
        pl.kernel,
        out_type=jax.ShapeDtypeStruct((2 * rows, 16), jnp.float32),
        mesh=mesh,
        compiler_params=pltpu.CompilerParams(use_tc_tiling_on_sc=False),
        scratch_types=[
            pltpu.VMEM((GS, CH), jnp.int32),
            pltpu.VMEM((GS, CH), jnp.int32),
            pltpu.VMEM((GS, CH, 16), jnp.float32),
            pltpu.VMEM((zb, 16), jnp.float32),
            pltpu.VMEM_SHARED((rows, 16), jnp.float32),
            pltpu.SemaphoreType.DMA,
            pltpu.SemaphoreType.DMA,
        ],
    )
    def agg(y_hbm, src_hbm, dst_hbm, out_hbm, sidx, didx, rv, zbuf, acc,
            gsem, ssem):
        c = lax.axis_index("c")
        s = lax.axis_index("s")
        wid = s * NC + c
        tile_base = s * rpt

        def zrow(i, _):
            zbuf[i, :] = jnp.zeros((16,), jnp.float32)
            return 0
        lax.fori_loop(0, zb, zrow, 0)

        def zacc(k, _):
            pltpu.sync_copy(zbuf, acc.at[pl.ds(tile_base + k * zb, zb)])
            return 0
        lax.fori_loop(0, 8, zacc, 0)
        plsc.subcore_barrier()

        chunk0 = wid * chunks_per_tile

        def group(g, _):
            r0 = chunk0 + g * GS
            pltpu.sync_copy(src_hbm.at[pl.ds(r0, GS)], sidx)
            pltpu.sync_copy(dst_hbm.at[pl.ds(r0, GS)], didx)
            gds = [pltpu.async_copy(y_hbm.at[sidx.at[j]], rv.at[j], gsem)
                   for j in range(GS)]
            for d in gds:
                d.wait()
            sds = [pltpu.async_copy(rv.at[j], acc.at[didx.at[j]], ssem,
                                    add=True)
                   for j in range(GS)]
            for d in sds:
                d.wait()
            return 0
        lax.fori_loop(0, n_groups, group, 0)
        plsc.subcore_barrier()

        pltpu.sync_copy(acc.at[pl.ds(tile_base, rpt)],
                        out_hbm.at[pl.ds(c * rows + tile_base, rpt)])

    del n_chunk_rows
    return agg


def _make_agg32(rows, chunks_per_tile):
    gs = 4
    rpt = rows // NS
    zb = 56
    n_zc = rpt // zb
    n_groups = chunks_per_tile // gs

    mesh = plsc.VectorSubcoreMesh(core_axis_name="c", subcore_axis_name="s")

    @functools.partial(
        pl.kernel,
        out_type=jax.ShapeDtypeStruct((2 * rows, 32), jnp.float32),
        mesh=mesh,
        compiler_params=pltpu.CompilerParams(use_tc_tiling_on_sc=False),
        scratch_types=[
            pltpu.VMEM((gs, CH), jnp.int32),
            pltpu.VMEM((gs, CH), jnp.int32),
            pltpu.VMEM((gs, CH, 32), jnp.float32),
            pltpu.VMEM((zb, 32), jnp.float32),
            pltpu.VMEM_SHARED((rows, 32), jnp.float32),
            pltpu.SemaphoreType.DMA,
            pltpu.SemaphoreType.DMA,
        ],
    )
    def agg(ylo_hbm, yhi_hbm, src_hbm, dst_hbm, out_hbm, sidx, didx, rv,
            zbuf, acc, gsem, ssem):
        c = lax.axis_index("c")
        s = lax.axis_index("s")
        tile_base = s * rpt

        def zrow(i, _):
            zbuf[i, 0:16] = jnp.zeros((16,), jnp.float32)
            zbuf[i, 16:32] = jnp.zeros((16,), jnp.float32)
            return 0
        lax.fori_loop(0, zb, zrow, 0)

        def zacc(k, _):
            pltpu.sync_copy(zbuf, acc.at[pl.ds(tile_base + k * zb, zb)])
            return 0
        lax.fori_loop(0, n_zc, zacc, 0)
        plsc.subcore_barrier()

        chunk0 = s * chunks_per_tile

        def body(y_hbm):
            def group(g, _):
                r0 = chunk0 + g * gs
                pltpu.sync_copy(src_hbm.at[pl.ds(r0, gs)], sidx)
                pltpu.sync_copy(dst_hbm.at[pl.ds(r0, gs)], didx)
                gds = [pltpu.async_copy(y_hbm.at[sidx.at[j]], rv.at[j], gsem)
                       for j in range(gs)]
                for d in gds:
                    d.wait()
                sds = [pltpu.async_copy(rv.at[j], acc.at[didx.at[j]], ssem,
                                        add=True)
                       for j in range(gs)]
                for d in sds:
                    d.wait()
                return 0
            lax.fori_loop(0, n_groups, group, 0)

        @pl.when(c == 0)
        def _():
            body(ylo_hbm)

        @pl.when(c == 1)
        def _():
            body(yhi_hbm)

        plsc.subcore_barrier()
        pltpu.sync_copy(acc.at[pl.ds(tile_base, rpt)],
                        out_hbm.at[pl.ds(c * rows + tile_base, rpt)])

    return agg


def _row_spec(br, d):
    return pl.BlockSpec((br, d), lambda i: (i, 0))


def _full_spec(shape):
    n = len(shape)
    return pl.BlockSpec(shape, lambda i: (0,) * n)


def _premod1_body(x_ref, wm_ref, bm_ref, y_ref):
    xb = x_ref[...]
    m = jax.nn.sigmoid(
        jnp.dot(xb, wm_ref[...].T, preferred_element_type=jnp.float32)
        + bm_ref[...])
    xm = xb * m
    br = xb.shape[0]
    y_ref[...] = jnp.concatenate(
        [xm, jnp.ones((br, 1), jnp.float32), jnp.zeros((br, 12), jnp.float32)],
        axis=1)


def _post1_body(ma_ref, mb_ref, y1_ref, wl_ref, bl_ref, wm_ref, bm_ref,
                ylo_ref, yhi_ref, rinv_ref):
    msg = ma_ref[...] + mb_ref[...] + y1_ref[...]
    rinv = 1.0 / msg[:, 3:4]
    aggr = msg[:, 0:3] * rinv
    h1 = jax.nn.relu(
        jnp.dot(aggr, wl_ref[...].T, preferred_element_type=jnp.float32)
        + bl_ref[...])
    m2 = jax.nn.sigmoid(
        jnp.dot(h1, wm_ref[...].T, preferred_element_type=jnp.float32)
        + bm_ref[...])
    xm2 = h1 * m2
    ylo_ref[...] = xm2[:, 0:32]
    yhi_ref[...] = xm2[:, 32:64]
    rinv_ref[...] = rinv


def _post2_body(ma_ref, mb_ref, ylo_ref, yhi_ref, rinv_ref, wl_ref, bl_ref,
                wm_ref, bm_ref, wl3_ref, y3_ref):
    rinv = rinv_ref[...]
    msg = jnp.concatenate(
        [ma_ref[...] + ylo_ref[...], mb_ref[...] + yhi_ref[...]], axis=1)
    aggr = msg * rinv
    h2 = jax.nn.relu(
        jnp.dot(aggr, wl_ref[...].T, preferred_element_type=jnp.float32)
        + bl_ref[...])
    m3 = jax.nn.sigmoid(
        jnp.dot(h2, wm_ref[...].T, preferred_element_type=jnp.float32)
        + bm_ref[...])
    xm3 = h2 * m3
    y3 = jnp.dot(xm3, wl3_ref[...].T, preferred_element_type=jnp.float32)
    br = y3.shape[0]
    y3_ref[...] = jnp.concatenate(
        [y3, jnp.zeros((br, 13), jnp.float32)], axis=1)


def _post3_body(ma_ref, mb_ref, y3_ref, rinv_ref, bl_ref, out_ref):
    msg = ma_ref[...] + mb_ref[...] + y3_ref[...]
    out_ref[...] = msg[:, 0:3] * rinv_ref[...] + bl_ref[...]


def kernel(x, edge_index, W1_lin, b1_lin, W1_mod, b1_mod,
           W2_lin, b2_lin, W2_mod, b2_mod,
           W3_lin, b3_lin, W3_mod, b3_mod):
    n = x.shape[0]
    e = edge_index.shape[1]
    rows = _round_up(n + 1, 256)
    epad = _round_up(e, NC * NS * CH * GS)
    ndum = rows - n

    xp = jnp.pad(x, ((0, rows - n), (0, 0)))
    pad_idx = (n + (jnp.arange(epad - e, dtype=jnp.int32) % ndum))
    srcp = jnp.concatenate([edge_index[0], pad_idx]).reshape(epad // CH, CH)
    dstp = jnp.concatenate([edge_index[1], pad_idx]).reshape(epad // CH, CH)

    br = 256
    grid = (rows // br,)
    cpt1 = epad // (NC * NS * CH)
    cpt2 = epad // (NS * CH)

    agg16 = _make_agg16(rows, epad // CH, cpt1)
    agg32 = _make_agg32(rows, cpt2)

    y1p = pl.pallas_call(
        _premod1_body,
        grid=grid,
        in_specs=[_row_spec(br, 3), _full_spec((3, 3)), _full_spec((1, 3))],
        out_specs=_row_spec(br, 16),
        out_shape=jax.ShapeDtypeStruct((rows, 16), jnp.float32),
    )(xp, W1_mod, b1_mod.reshape(1, 3))

    m1 = agg16(y1p, srcp, dstp)

    y2lo, y2hi, rinv = pl.pallas_call(
        _post1_body,
        grid=grid,
        in_specs=[_row_spec(br, 16), _row_spec(br, 16), _row_spec(br, 16),
                  _full_spec((64, 3)), _full_spec((1, 64)),
                  _full_spec((64, 64)), _full_spec((1, 64))],
        out_specs=[_row_spec(br, 32), _row_spec(br, 32), _row_spec(br, 1)],
        out_shape=[jax.ShapeDtypeStruct((rows, 32), jnp.float32),
                   jax.ShapeDtypeStruct((rows, 32), jnp.float32),
                   jax.ShapeDtypeStruct((rows, 1), jnp.float32)],
    )(m1[:rows], m1[rows:], y1p, W1_lin, b1_lin.reshape(1, 64),
      W2_mod, b2_mod.reshape(1, 64))

    m2 = agg32(y2lo, y2hi, srcp, dstp)

    y3p = pl.pallas_call(
        _post2_body,
        grid=grid,
        in_specs=[_row_spec(br, 32), _row_spec(br, 32), _row_spec(br, 32),
                  _row_spec(br, 32), _row_spec(br, 1),
                  _full_spec((128, 64)), _full_spec((1, 128)),
                  _full_spec((128, 128)), _full_spec((1, 128)),
                  _full_spec((3, 128))],
        out_specs=_row_spec(br, 16),
        out_shape=jax.ShapeDtypeStruct((rows, 16), jnp.float32),
    )(m2[:rows], m2[rows:], y2lo, y2hi, rinv,
      W2_lin, b2_lin.reshape(1, 128), W3_mod, b3_mod.reshape(1, 128), W3_lin)

    m3 = agg16(y3p, srcp, dstp)

    out = pl.pallas_call(
        _post3_body,
        grid=grid,
        in_specs=[_row_spec(br, 16), _row_spec(br, 16), _row_spec(br, 16),
                  _row_spec(br, 1), _full_spec((1, 3))],
        out_specs=_row_spec(br, 3),
        out_shape=jax.ShapeDtypeStruct((rows, 3), jnp.float32),
    )(m3[:rows], m3[rows:], y3p, rinv, b3_lin.reshape(1, 3))

    return out[:n]


# speedup vs baseline: 23.4488x; 1.3484x over previous
import functools

import jax
import jax.numpy as jnp
from jax import lax
from jax.experimental import pallas as pl
from jax.experimental.pallas import tpu as pltpu
from jax.experimental.pallas import tpu_sc as plsc

NC = 2
NS = 16
CH = 128
GS = 8


def _round_up(a, m):
    return (a + m - 1) // m * m


def _make_agg16(rows, n_chunk_rows, chunks_per_tile):
    rpt = rows // NS
    zb = rpt // 8
    n_groups = chunks_per_tile // GS

    mesh = plsc.VectorSubcoreMesh(core_axis_name="c", subcore_axis_name="s")

    @functools.partial(
        pl.kernel,
        out_type=jax.ShapeDtypeStruct((2 * rows, 16), jnp.float32),
        mesh=mesh,
        compiler_params=pltpu.CompilerParams(use_tc_tiling_on_sc=False),
        scratch_types=[
            pltpu.VMEM((GS, CH), jnp.int32),
            pltpu.VMEM((GS, CH), jnp.int32),
            pltpu.VMEM((GS, CH, 16), jnp.float32),
            pltpu.VMEM((zb, 16), jnp.float32),
            pltpu.VMEM_SHARED((rows, 16), jnp.float32),
            pltpu.SemaphoreType.DMA,
            pltpu.SemaphoreType.DMA,
        ],
    )
    def agg(y_hbm, src_hbm, dst_hbm, out_hbm, sidx, didx, rv, zbuf, acc,
            gsem, ssem):
        c = lax.axis_index("c")
        s = lax.axis_index("s")
        wid = s * NC + c
        tile_base = s * rpt

        def zrow(i, _):
            zbuf[i, :] = jnp.zeros((16,), jnp.float32)
            return 0
        lax.fori_loop(0, zb, zrow, 0)

        def zacc(k, _):
            pltpu.sync_copy(zbuf, acc.at[pl.ds(tile_base + k * zb, zb)])
            return 0
        lax.fori_loop(0, 8, zacc, 0)
        plsc.subcore_barrier()

        chunk0 = wid * chunks_per_tile

        def group(g, _):
            r0 = chunk0 + g * GS
            pltpu.sync_copy(src_hbm.at[pl.ds(r0, GS)], sidx)
            pltpu.sync_copy(dst_hbm.at[pl.ds(r0, GS)], didx)
            gds = [pltpu.async_copy(y_hbm.at[sidx.at[j]], rv.at[j], gsem)
                   for j in range(GS)]
            for d in gds:
                d.wait()
            sds = [pltpu.async_copy(rv.at[j], acc.at[didx.at[j]], ssem,
                                    add=True)
                   for j in range(GS)]
            for d in sds:
                d.wait()
            return 0
        lax.fori_loop(0, n_groups, group, 0)
        plsc.subcore_barrier()

        pltpu.sync_copy(acc.at[pl.ds(tile_base, rpt)],
                        out_hbm.at[pl.ds(c * rows + tile_base, rpt)])

    del n_chunk_rows
    return agg


def _make_agg32(rows, chunks_per_tile):
    gs = 4
    rpt = rows // NS
    zb = 56
    n_zc = rpt // zb
    n_groups = chunks_per_tile // gs

    mesh = plsc.VectorSubcoreMesh(core_axis_name="c", subcore_axis_name="s")

    @functools.partial(
        pl.kernel,
        out_type=jax.ShapeDtypeStruct((2 * rows, 32), jnp.float32),
        mesh=mesh,
        compiler_params=pltpu.CompilerParams(use_tc_tiling_on_sc=False),
        scratch_types=[
            pltpu.VMEM((gs, CH), jnp.int32),
            pltpu.VMEM((gs, CH), jnp.int32),
            pltpu.VMEM((gs, CH, 32), jnp.float32),
            pltpu.VMEM((zb, 32), jnp.float32),
            pltpu.VMEM_SHARED((rows, 32), jnp.float32),
            pltpu.SemaphoreType.DMA,
            pltpu.SemaphoreType.DMA,
        ],
    )
    def agg(ylo_hbm, yhi_hbm, src_hbm, dst_hbm, out_hbm, sidx, didx, rv,
            zbuf, acc, gsem, ssem):
        c = lax.axis_index("c")
        s = lax.axis_index("s")
        tile_base = s * rpt

        def zrow(i, _):
            zbuf[i, 0:16] = jnp.zeros((16,), jnp.float32)
            zbuf[i, 16:32] = jnp.zeros((16,), jnp.float32)
            return 0
        lax.fori_loop(0, zb, zrow, 0)

        def zacc(k, _):
            pltpu.sync_copy(zbuf, acc.at[pl.ds(tile_base + k * zb, zb)])
            return 0
        lax.fori_loop(0, n_zc, zacc, 0)
        plsc.subcore_barrier()

        chunk0 = s * chunks_per_tile

        def body(y_hbm):
            def group(g, _):
                r0 = chunk0 + g * gs
                pltpu.sync_copy(src_hbm.at[pl.ds(r0, gs)], sidx)
                pltpu.sync_copy(dst_hbm.at[pl.ds(r0, gs)], didx)
                gds = [pltpu.async_copy(y_hbm.at[sidx.at[j]], rv.at[j], gsem)
                       for j in range(gs)]
                for d in gds:
                    d.wait()
                sds = [pltpu.async_copy(rv.at[j], acc.at[didx.at[j]], ssem,
                                        add=True)
                       for j in range(gs)]
                for d in sds:
                    d.wait()
                return 0
            lax.fori_loop(0, n_groups, group, 0)

        @pl.when(c == 0)
        def _():
            body(ylo_hbm)

        @pl.when(c == 1)
        def _():
            body(yhi_hbm)

        plsc.subcore_barrier()
        pltpu.sync_copy(acc.at[pl.ds(tile_base, rpt)],
                        out_hbm.at[pl.ds(c * rows + tile_base, rpt)])

    return agg


def _row_spec(br, d):
    return pl.BlockSpec((br, d), lambda i: (i, 0))


def _row_spec_off(br, d, off_blocks):
    return pl.BlockSpec((br, d), lambda i: (i + off_blocks, 0))


def _full_spec(shape):
    n = len(shape)
    return pl.BlockSpec(shape, lambda i: (0,) * n)


def _premod1_body(x_ref, wm_ref, bm_ref, y_ref):
    xb = x_ref[...]
    m = jax.nn.sigmoid(
        jnp.dot(xb, wm_ref[...].T, preferred_element_type=jnp.float32)
        + bm_ref[...])
    xm = xb * m
    br = xb.shape[0]
    y_ref[...] = jnp.concatenate(
        [xm, jnp.ones((br, 1), jnp.float32), jnp.zeros((br, 12), jnp.float32)],
        axis=1)


def _post1_body(ma_ref, mb_ref, y1_ref, wl_ref, bl_ref, wm_ref, bm_ref,
                ylo_ref, yhi_ref, rinv_ref):
    msg = ma_ref[...] + mb_ref[...] + y1_ref[...]
    rinv = 1.0 / msg[:, 3:4]
    aggr = msg[:, 0:3] * rinv
    h1 = jax.nn.relu(
        jnp.dot(aggr, wl_ref[...].T, preferred_element_type=jnp.float32)
        + bl_ref[...])
    m2 = jax.nn.sigmoid(
        jnp.dot(h1, wm_ref[...].T, preferred_element_type=jnp.float32)
        + bm_ref[...])
    xm2 = h1 * m2
    ylo_ref[...] = xm2[:, 0:32]
    yhi_ref[...] = xm2[:, 32:64]
    rinv_ref[...] = rinv


def _post2_body(ma_ref, mb_ref, ylo_ref, yhi_ref, rinv_ref, wl_ref, bl_ref,
                wm_ref, bm_ref, wl3_ref, y3_ref):
    rinv = rinv_ref[...]
    msg = jnp.concatenate(
        [ma_ref[...] + ylo_ref[...], mb_ref[...] + yhi_ref[...]], axis=1)
    aggr = msg * rinv
    h2 = jax.nn.relu(
        jnp.dot(aggr, wl_ref[...].T, preferred_element_type=jnp.float32)
        + bl_ref[...])
    m3 = jax.nn.sigmoid(
        jnp.dot(h2, wm_ref[...].T, preferred_element_type=jnp.float32)
        + bm_ref[...])
    xm3 = h2 * m3
    y3 = jnp.dot(xm3, wl3_ref[...].T, preferred_element_type=jnp.float32)
    br = y3.shape[0]
    y3_ref[...] = jnp.concatenate(
        [y3, jnp.zeros((br, 13), jnp.float32)], axis=1)


def _post3_body(ma_ref, mb_ref, y3_ref, rinv_ref, bl_ref, out_ref):
    msg = ma_ref[...] + mb_ref[...] + y3_ref[...]
    out_ref[...] = msg[:, 0:3] * rinv_ref[...] + bl_ref[...]


def kernel(x, edge_index, W1_lin, b1_lin, W1_mod, b1_mod,
           W2_lin, b2_lin, W2_mod, b2_mod,
           W3_lin, b3_lin, W3_mod, b3_mod):
    n = x.shape[0]
    e = edge_index.shape[1]
    rows = _round_up(n + 1, 256)
    epad = _round_up(e, NC * NS * CH * GS)
    ndum = rows - n

    xp = jnp.pad(x, ((0, rows - n), (0, 0)))
    pad_idx = (n + (jnp.arange(epad - e, dtype=jnp.int32) % ndum))
    srcp = jnp.concatenate([edge_index[0], pad_idx]).reshape(epad // CH, CH)
    dstp = jnp.concatenate([edge_index[1], pad_idx]).reshape(epad // CH, CH)

    br = 3136
    grid = (rows // br,)
    cpt1 = epad // (NC * NS * CH)
    cpt2 = epad // (NS * CH)

    agg16 = _make_agg16(rows, epad // CH, cpt1)
    agg32 = _make_agg32(rows, cpt2)

    y1p = pl.pallas_call(
        _premod1_body,
        grid=grid,
        in_specs=[_row_spec(br, 3), _full_spec((3, 3)), _full_spec((1, 3))],
        out_specs=_row_spec(br, 16),
        out_shape=jax.ShapeDtypeStruct((rows, 16), jnp.float32),
    )(xp, W1_mod, b1_mod.reshape(1, 3))

    m1 = agg16(y1p, srcp, dstp)

    y2lo, y2hi, rinv = pl.pallas_call(
        _post1_body,
        grid=grid,
        in_specs=[_row_spec(br, 16), _row_spec_off(br, 16, rows // br),
                  _row_spec(br, 16),
                  _full_spec((64, 3)), _full_spec((1, 64)),
                  _full_spec((64, 64)), _full_spec((1, 64))],
        out_specs=[_row_spec(br, 32), _row_spec(br, 32), _row_spec(br, 1)],
        out_shape=[jax.ShapeDtypeStruct((rows, 32), jnp.float32),
                   jax.ShapeDtypeStruct((rows, 32), jnp.float32),
                   jax.ShapeDtypeStruct((rows, 1), jnp.float32)],
    )(m1, m1, y1p, W1_lin, b1_lin.reshape(1, 64),
      W2_mod, b2_mod.reshape(1, 64))

    m2 = agg32(y2lo, y2hi, srcp, dstp)

    y3p = pl.pallas_call(
        _post2_body,
        grid=grid,
        in_specs=[_row_spec(br, 32), _row_spec_off(br, 32, rows // br),
                  _row_spec(br, 32), _row_spec(br, 32), _row_spec(br, 1),
                  _full_spec((128, 64)), _full_spec((1, 128)),
                  _full_spec((128, 128)), _full_spec((1, 128)),
                  _full_spec((3, 128))],
        out_specs=_row_spec(br, 16),
        out_shape=jax.ShapeDtypeStruct((rows, 16), jnp.float32),
    )(m2, m2, y2lo, y2hi, rinv,
      W2_lin, b2_lin.reshape(1, 128), W3_mod, b3_mod.reshape(1, 128), W3_lin)

    m3 = agg16(y3p, srcp, dstp)

    out = pl.pallas_call(
        _post3_body,
        grid=grid,
        in_specs=[_row_spec(br, 16), _row_spec_off(br, 16, rows // br),
                  _row_spec(br, 16),
                  _row_spec(br, 1), _full_spec((1, 3))],
        out_specs=_row_spec(br, 3),
        out_shape=jax.ShapeDtypeStruct((rows, 3), jnp.float32),
    )(m3, m3, y3p, rinv, b3_lin.reshape(1, 3))

    return out[:n]


# speedup vs baseline: 30.9342x; 1.3192x over previous
import functools

import jax
import jax.numpy as jnp
from jax import lax
from jax.experimental import pallas as pl
from jax.experimental.pallas import tpu as pltpu
from jax.experimental.pallas import tpu_sc as plsc

NC = 2
NS = 16
CH = 128
GS = 8


def _round_up(a, m):
    return (a + m - 1) // m * m


def _make_agg(rows, d, chunks_per_tile, wv, ib, edge_split):
    rpt = rows // NS
    zb = 56
    n_zc = rpt // zb
    nw = chunks_per_tile // wv
    nb = nw // ib
    assert nw % ib == 0 and rpt % zb == 0 and chunks_per_tile % wv == 0

    mesh = plsc.VectorSubcoreMesh(core_axis_name="c", subcore_axis_name="s")

    @functools.partial(
        pl.kernel,
        out_type=jax.ShapeDtypeStruct((2 * rows, d), jnp.float32),
        mesh=mesh,
        compiler_params=pltpu.CompilerParams(use_tc_tiling_on_sc=False),
        scratch_types=[
            pltpu.VMEM((2, ib * wv, CH), jnp.int32),
            pltpu.VMEM((2, ib * wv, CH), jnp.int32),
            pltpu.VMEM((2, wv, CH, d), jnp.float32),
            pltpu.VMEM((zb, d), jnp.float32),
            pltpu.VMEM_SHARED((rows, d), jnp.float32),
            pltpu.SemaphoreType.DMA,
            pltpu.SemaphoreType.DMA,
            pltpu.SemaphoreType.DMA,
        ],
    )
    def agg(ylo_hbm, yhi_hbm, src_hbm, dst_hbm, out_hbm, sidx, didx, rv,
            zbuf, acc, gsem, ssem, isem):
        c = lax.axis_index("c")
        s = lax.axis_index("s")
        wid = s * NC + c
        tile_base = s * rpt

        def zrow(i, _):
            for k in range(d // 16):
                zbuf[i, 16 * k:16 * (k + 1)] = jnp.zeros((16,), jnp.float32)
            return 0
        lax.fori_loop(0, zb, zrow, 0)

        def zacc(k, _):
            pltpu.sync_copy(zbuf, acc.at[pl.ds(tile_base + k * zb, zb)])
            return 0
        lax.fori_loop(0, n_zc, zacc, 0)
        plsc.subcore_barrier()

        chunk0 = (wid if edge_split else s) * chunks_per_tile

        def idx_load(b, slot):
            r0 = chunk0 + b * ib * wv
            pltpu.async_copy(src_hbm.at[pl.ds(r0, ib * wv)], sidx.at[slot],
                             isem)
            pltpu.async_copy(dst_hbm.at[pl.ds(r0, ib * wv)], didx.at[slot],
                             isem)

        def wait_idx():
            for _ in range(2):
                pltpu.make_async_copy(src_hbm.at[pl.ds(0, ib * wv)],
                                      sidx.at[0], isem).wait()

        def body(y_hbm):
            def fire_g(g, p, q):
                w = lax.rem(g, ib) * wv
                for j in range(wv):
                    pltpu.async_copy(y_hbm.at[sidx.at[q, w + j]],
                                     rv.at[p, j], gsem)

            def wait_g():
                for j in range(wv):
                    pltpu.make_async_copy(y_hbm.at[pl.ds(0, CH)],
                                          rv.at[0, j], gsem).wait()

            def fire_s(g, p, q):
                w = lax.rem(g, ib) * wv
                for j in range(wv):
                    pltpu.async_copy(rv.at[p, j], acc.at[didx.at[q, w + j]],
                                     ssem, add=True)

            def wait_s():
                for j in range(wv):
                    pltpu.make_async_copy(y_hbm.at[pl.ds(0, CH)],
                                          rv.at[0, j], ssem).wait()

            idx_load(0, 0)
            wait_idx()
            fire_g(0, 0, 0)

            def step(g, _):
                p = lax.rem(g, 2)
                b = g // ib
                q = lax.rem(b, 2)
                wait_g()
                @pl.when(g >= 2)
                def _():
                    wait_s()
                fire_s(g - 1, 1 - p, lax.rem((g - 1) // ib, 2))
                @pl.when(lax.rem(g, ib) == 0)
                def _():
                    wait_idx()
                @pl.when(lax.rem(g, ib) == 1)
                def _():
                    @pl.when(b + 1 < nb)
                    def _():
                        idx_load(b + 1, lax.rem(b + 1, 2))
                fire_g(g, p, q)
                return 0
            lax.fori_loop(1, nw, step, 0)

            wait_g()
            if nw >= 2:
                wait_s()
            fire_s(nw - 1, lax.rem(nw - 1, 2), lax.rem((nw - 1) // ib, 2))
            wait_s()

        if edge_split:
            body(ylo_hbm)
        else:
            @pl.when(c == 0)
            def _():
                body(ylo_hbm)

            @pl.when(c == 1)
            def _():
                body(yhi_hbm)

        plsc.subcore_barrier()
        pltpu.sync_copy(acc.at[pl.ds(tile_base, rpt)],
                        out_hbm.at[pl.ds(c * rows + tile_base, rpt)])

    return agg


def _row_spec(br, d):
    return pl.BlockSpec((br, d), lambda i: (i, 0))


def _row_spec_off(br, d, off_blocks):
    return pl.BlockSpec((br, d), lambda i: (i + off_blocks, 0))


def _full_spec(shape):
    n = len(shape)
    return pl.BlockSpec(shape, lambda i: (0,) * n)


def _premod1_body(x_ref, wm_ref, bm_ref, y_ref):
    xb = x_ref[...]
    m = jax.nn.sigmoid(
        jnp.dot(xb, wm_ref[...].T, preferred_element_type=jnp.float32)
        + bm_ref[...])
    xm = xb * m
    br = xb.shape[0]
    y_ref[...] = jnp.concatenate(
        [xm, jnp.ones((br, 1), jnp.float32), jnp.zeros((br, 12), jnp.float32)],
        axis=1)


def _post1_body(ma_ref, mb_ref, y1_ref, wl_ref, bl_ref, wm_ref, bm_ref,
                ylo_ref, yhi_ref, rinv_ref):
    msg = ma_ref[...] + mb_ref[...] + y1_ref[...]
    rinv = 1.0 / msg[:, 3:4]
    aggr = msg[:, 0:3] * rinv
    h1 = jax.nn.relu(
        jnp.dot(aggr, wl_ref[...].T, preferred_element_type=jnp.float32)
        + bl_ref[...])
    m2 = jax.nn.sigmoid(
        jnp.dot(h1, wm_ref[...].T, preferred_element_type=jnp.float32)
        + bm_ref[...])
    xm2 = h1 * m2
    ylo_ref[...] = xm2[:, 0:32]
    yhi_ref[...] = xm2[:, 32:64]
    rinv_ref[...] = rinv


def _post2_body(ma_ref, mb_ref, ylo_ref, yhi_ref, rinv_ref, wl_ref, bl_ref,
                wm_ref, bm_ref, wl3_ref, y3_ref):
    rinv = rinv_ref[...]
    msg = jnp.concatenate(
        [ma_ref[...] + ylo_ref[...], mb_ref[...] + yhi_ref[...]], axis=1)
    aggr = msg * rinv
    h2 = jax.nn.relu(
        jnp.dot(aggr, wl_ref[...].T, preferred_element_type=jnp.float32)
        + bl_ref[...])
    m3 = jax.nn.sigmoid(
        jnp.dot(h2, wm_ref[...].T, preferred_element_type=jnp.float32)
        + bm_ref[...])
    xm3 = h2 * m3
    y3 = jnp.dot(xm3, wl3_ref[...].T, preferred_element_type=jnp.float32)
    br = y3.shape[0]
    y3_ref[...] = jnp.concatenate(
        [y3, jnp.zeros((br, 13), jnp.float32)], axis=1)


def _post3_body(ma_ref, mb_ref, y3_ref, rinv_ref, bl_ref, out_ref):
    msg = ma_ref[...] + mb_ref[...] + y3_ref[...]
    out_ref[...] = msg[:, 0:3] * rinv_ref[...] + bl_ref[...]


def kernel(x, edge_index, W1_lin, b1_lin, W1_mod, b1_mod,
           W2_lin, b2_lin, W2_mod, b2_mod,
           W3_lin, b3_lin, W3_mod, b3_mod):
    n = x.shape[0]
    e = edge_index.shape[1]
    rows = _round_up(n + 1, 256)
    epad = _round_up(e, NC * NS * CH * GS)
    ndum = rows - n

    xp = jnp.pad(x, ((0, rows - n), (0, 0)))
    pad_idx = (n + (jnp.arange(epad - e, dtype=jnp.int32) % ndum))
    srcp = jnp.concatenate([edge_index[0], pad_idx]).reshape(epad // CH, CH)
    dstp = jnp.concatenate([edge_index[1], pad_idx]).reshape(epad // CH, CH)

    br = 3136
    grid = (rows // br,)
    cpt1 = epad // (NC * NS * CH)
    cpt2 = epad // (NS * CH)

    agg16 = _make_agg(rows, 16, cpt1, wv=8, ib=7, edge_split=True)
    agg32 = _make_agg(rows, 32, cpt2, wv=2, ib=7, edge_split=False)

    y1p = pl.pallas_call(
        _premod1_body,
        grid=grid,
        in_specs=[_row_spec(br, 3), _full_spec((3, 3)), _full_spec((1, 3))],
        out_specs=_row_spec(br, 16),
        out_shape=jax.ShapeDtypeStruct((rows, 16), jnp.float32),
    )(xp, W1_mod, b1_mod.reshape(1, 3))

    m1 = agg16(y1p, y1p, srcp, dstp)

    y2lo, y2hi, rinv = pl.pallas_call(
        _post1_body,
        grid=grid,
        in_specs=[_row_spec(br, 16), _row_spec_off(br, 16, rows // br),
                  _row_spec(br, 16),
                  _full_spec((64, 3)), _full_spec((1, 64)),
                  _full_spec((64, 64)), _full_spec((1, 64))],
        out_specs=[_row_spec(br, 32), _row_spec(br, 32), _row_spec(br, 1)],
        out_shape=[jax.ShapeDtypeStruct((rows, 32), jnp.float32),
                   jax.ShapeDtypeStruct((rows, 32), jnp.float32),
                   jax.ShapeDtypeStruct((rows, 1), jnp.float32)],
    )(m1, m1, y1p, W1_lin, b1_lin.reshape(1, 64),
      W2_mod, b2_mod.reshape(1, 64))

    m2 = agg32(y2lo, y2hi, srcp, dstp)

    y3p = pl.pallas_call(
        _post2_body,
        grid=grid,
        in_specs=[_row_spec(br, 32), _row_spec_off(br, 32, rows // br),
                  _row_spec(br, 32), _row_spec(br, 32), _row_spec(br, 1),
                  _full_spec((128, 64)), _full_spec((1, 128)),
                  _full_spec((128, 128)), _full_spec((1, 128)),
                  _full_spec((3, 128))],
        out_specs=_row_spec(br, 16),
        out_shape=jax.ShapeDtypeStruct((rows, 16), jnp.float32),
    )(m2, m2, y2lo, y2hi, rinv,
      W2_lin, b2_lin.reshape(1, 128), W3_mod, b3_mod.reshape(1, 128), W3_lin)

    m3 = agg16(y3p, y3p, srcp, dstp)

    out = pl.pallas_call(
        _post3_body,
        grid=grid,
        in_specs=[_row_spec(br, 16), _row_spec_off(br, 16, rows // br),
                  _row_spec(br, 16),
                  _row_spec(br, 1), _full_spec((1, 3))],
        out_specs=_row_spec(br, 3),
        out_shape=jax.ShapeDtypeStruct((rows, 3), jnp.float32),
    )(m3, m3, y3p, rinv, b3_lin.reshape(1, 3))

    return out[:n]


# speedup vs baseline: 36.9032x; 1.1930x over previous
import functools

import jax
import jax.numpy as jnp
from jax import lax
from jax.experimental import pallas as pl
from jax.experimental.pallas import tpu as pltpu
from jax.experimental.pallas import tpu_sc as plsc

NC = 2
NS = 16
CH = 128


def _round_up(a, m):
    return (a + m - 1) // m * m


def _make_agg(rows, d, chunks_per_tile, wv, ib, edge_split):
    rpt = rows // NS
    zb = 56
    n_zc = rpt // zb
    nw = chunks_per_tile // wv
    nb = nw // ib
    assert nw % ib == 0 and rpt % zb == 0 and chunks_per_tile % wv == 0

    mesh = plsc.VectorSubcoreMesh(core_axis_name="c", subcore_axis_name="s")

    @functools.partial(
        pl.kernel,
        out_type=jax.ShapeDtypeStruct((2 * rows, d), jnp.float32),
        mesh=mesh,
        compiler_params=pltpu.CompilerParams(use_tc_tiling_on_sc=False),
        scratch_types=[
            pltpu.VMEM((2, ib * wv, CH), jnp.int32),
            pltpu.VMEM((2, ib * wv, CH), jnp.int32),
            pltpu.VMEM((2, wv, CH, d), jnp.float32),
            pltpu.VMEM((zb, d), jnp.float32),
            pltpu.VMEM_SHARED((rows, d), jnp.float32),
            pltpu.SemaphoreType.DMA,
            pltpu.SemaphoreType.DMA,
            pltpu.SemaphoreType.DMA,
        ],
    )
    def agg(ylo_hbm, yhi_hbm, src_hbm, dst_hbm, out_hbm, sidx, didx, rv,
            zbuf, acc, gsem, ssem, isem):
        c = lax.axis_index("c")
        s = lax.axis_index("s")
        wid = s * NC + c
        tile_base = s * rpt

        def zrow(i, _):
            for k in range(d // 16):
                zbuf[i, 16 * k:16 * (k + 1)] = jnp.zeros((16,), jnp.float32)
            return 0
        lax.fori_loop(0, zb, zrow, 0)

        def zacc(k, _):
            pltpu.sync_copy(zbuf, acc.at[pl.ds(tile_base + k * zb, zb)])
            return 0
        lax.fori_loop(0, n_zc, zacc, 0)
        plsc.subcore_barrier()

        chunk0 = (wid if edge_split else s) * chunks_per_tile

        def idx_load(b, slot):
            r0 = chunk0 + b * ib * wv
            pltpu.async_copy(src_hbm.at[pl.ds(r0, ib * wv)], sidx.at[slot],
                             isem)
            pltpu.async_copy(dst_hbm.at[pl.ds(r0, ib * wv)], didx.at[slot],
                             isem)

        def wait_idx():
            for _ in range(2):
                pltpu.make_async_copy(src_hbm.at[pl.ds(0, ib * wv)],
                                      sidx.at[0], isem).wait()

        def body(y_hbm):
            def fire_g(g, p, q):
                w = lax.rem(g, ib) * wv
                for j in range(wv):
                    pltpu.async_copy(y_hbm.at[sidx.at[q, w + j]],
                                     rv.at[p, j], gsem)

            def wait_g():
                for j in range(wv):
                    pltpu.make_async_copy(y_hbm.at[pl.ds(0, CH)],
                                          rv.at[0, j], gsem).wait()

            def fire_s(g, p, q):
                w = lax.rem(g, ib) * wv
                for j in range(wv):
                    pltpu.async_copy(rv.at[p, j], acc.at[didx.at[q, w + j]],
                                     ssem, add=True)

            def wait_s():
                for j in range(wv):
                    pltpu.make_async_copy(y_hbm.at[pl.ds(0, CH)],
                                          rv.at[0, j], ssem).wait()

            idx_load(0, 0)
            wait_idx()
            fire_g(0, 0, 0)

            def step(g, _):
                p = lax.rem(g, 2)
                b = g // ib
                q = lax.rem(b, 2)
                wait_g()
                @pl.when(g >= 2)
                def _():
                    wait_s()
                fire_s(g - 1, 1 - p, lax.rem((g - 1) // ib, 2))
                @pl.when(lax.rem(g, ib) == 0)
                def _():
                    wait_idx()
                @pl.when(lax.rem(g, ib) == 1)
                def _():
                    @pl.when(b + 1 < nb)
                    def _():
                        idx_load(b + 1, lax.rem(b + 1, 2))
                fire_g(g, p, q)
                return 0
            lax.fori_loop(1, nw, step, 0)

            wait_g()
            if nw >= 2:
                wait_s()
            fire_s(nw - 1, lax.rem(nw - 1, 2), lax.rem((nw - 1) // ib, 2))
            wait_s()

        if edge_split:
            body(ylo_hbm)
        else:
            @pl.when(c == 0)
            def _():
                body(ylo_hbm)

            @pl.when(c == 1)
            def _():
                body(yhi_hbm)

        plsc.subcore_barrier()
        pltpu.sync_copy(acc.at[pl.ds(tile_base, rpt)],
                        out_hbm.at[pl.ds(c * rows + tile_base, rpt)])

    return agg


def _row_spec(br, d):
    return pl.BlockSpec((br, d), lambda i: (i, 0))


def _row_spec_off(br, d, off_blocks):
    return pl.BlockSpec((br, d), lambda i: (i + off_blocks, 0))


def _full_spec(shape):
    n = len(shape)
    return pl.BlockSpec(shape, lambda i: (0,) * n)


def _premod1_body(x_ref, wm_ref, bm_ref, sp_ref, ones_ref, y_ref):
    xb = x_ref[...]
    m = jax.nn.sigmoid(
        jnp.dot(xb, wm_ref[...], preferred_element_type=jnp.float32)
        + bm_ref[...])
    xm = xb * m
    y_ref[...] = (jnp.dot(xm, sp_ref[...], preferred_element_type=jnp.float32)
                  + ones_ref[...])


def _post1_body(ma_ref, mb_ref, y1_ref, s16_ref, wla_ref, wlb_ref,
                bla_ref, blb_ref, wmaa_ref, wmab_ref, wmba_ref, wmbb_ref,
                bma_ref, bmb_ref, ylo_ref, yhi_ref, rinv_ref):
    msg = ma_ref[...] + mb_ref[...] + y1_ref[...]
    v = jnp.dot(msg, s16_ref[...], preferred_element_type=jnp.float32)
    rinv = 1.0 / v
    aggr = msg * rinv
    h1lo = jax.nn.relu(
        jnp.dot(aggr, wla_ref[...], preferred_element_type=jnp.float32)
        + bla_ref[...])
    h1hi = jax.nn.relu(
        jnp.dot(aggr, wlb_ref[...], preferred_element_type=jnp.float32)
        + blb_ref[...])
    m2lo = jax.nn.sigmoid(
        jnp.dot(h1lo, wmaa_ref[...], preferred_element_type=jnp.float32)
        + jnp.dot(h1hi, wmba_ref[...], preferred_element_type=jnp.float32)
        + bma_ref[...])
    m2hi = jax.nn.sigmoid(
        jnp.dot(h1lo, wmab_ref[...], preferred_element_type=jnp.float32)
        + jnp.dot(h1hi, wmbb_ref[...], preferred_element_type=jnp.float32)
        + bmb_ref[...])
    ylo_ref[...] = h1lo * m2lo
    yhi_ref[...] = h1hi * m2hi
    rinv_ref[...] = rinv


def _post2_body(ma_ref, mb_ref, ylo_ref, yhi_ref, rinv_ref, r32_ref,
                wla_ref, wlb_ref, bl_ref, wm_ref, bm_ref, wl3_ref, y3_ref):
    rs32 = jnp.dot(rinv_ref[...], r32_ref[...],
                   preferred_element_type=jnp.float32)
    alo = (ma_ref[...] + ylo_ref[...]) * rs32
    ahi = (mb_ref[...] + yhi_ref[...]) * rs32
    h2 = jax.nn.relu(
        jnp.dot(alo, wla_ref[...], preferred_element_type=jnp.float32)
        + jnp.dot(ahi, wlb_ref[...], preferred_element_type=jnp.float32)
        + bl_ref[...])
    m3 = jax.nn.sigmoid(
        jnp.dot(h2.astype(jnp.bfloat16), wm_ref[...],
                preferred_element_type=jnp.float32)
        + bm_ref[...])
    xm3 = h2 * m3
    y3_ref[...] = jnp.dot(xm3, wl3_ref[...],
                          preferred_element_type=jnp.float32)


def _post3_body(ma_ref, mb_ref, y3_ref, rinv_ref, bl_ref, o_ref):
    msg = ma_ref[...] + mb_ref[...] + y3_ref[...]
    o_ref[...] = msg * rinv_ref[...] + bl_ref[...]


def _kron8(w):
    return jnp.kron(jnp.eye(8, dtype=jnp.float32), w)


def _tile8(b):
    return jnp.tile(b, 8).reshape(1, -1)


def kernel(x, edge_index, W1_lin, b1_lin, W1_mod, b1_mod,
           W2_lin, b2_lin, W2_mod, b2_mod,
           W3_lin, b3_lin, W3_mod, b3_mod):
    n = x.shape[0]
    e = edge_index.shape[1]
    rows = _round_up(n + 1, 256)
    pk = rows // 8
    epad = _round_up(e, NC * NS * CH * 8)
    ndum = rows - n

    x8 = jnp.pad(x, ((0, rows - n), (0, 0))).reshape(pk, 24)
    pad_idx = (n + (jnp.arange(epad - e, dtype=jnp.int32) % ndum))
    srcp = jnp.concatenate([edge_index[0], pad_idx]).reshape(epad // CH, CH)
    dstp = jnp.concatenate([edge_index[1], pad_idx]).reshape(epad // CH, CH)

    f32 = jnp.float32
    w1m = _kron8(W1_mod.T)
    b1m = _tile8(b1_mod)
    sp1 = _kron8(jnp.pad(jnp.eye(3, dtype=f32), ((0, 0), (0, 13))))
    lane = jnp.arange(128)
    ones3 = jnp.where(lane % 16 == 3, 1.0, 0.0).astype(f32).reshape(1, 128)
    s16 = _kron8((jnp.arange(16)[:, None] == 3).astype(f32)
                 * jnp.ones((16, 16), f32))
    w1la = _kron8(jnp.pad(W1_lin[:32].T, ((0, 13), (0, 0))))
    w1lb = _kron8(jnp.pad(W1_lin[32:].T, ((0, 13), (0, 0))))
    b1la = _tile8(b1_lin[:32])
    b1lb = _tile8(b1_lin[32:])
    w2maa = _kron8(W2_mod[:32, :32].T)
    w2mab = _kron8(W2_mod[32:, :32].T)
    w2mba = _kron8(W2_mod[:32, 32:].T)
    w2mbb = _kron8(W2_mod[32:, 32:].T)
    b2ma = _tile8(b2_mod[:32])
    b2mb = _tile8(b2_mod[32:])
    r32 = _kron8((jnp.arange(16)[:, None] == 0).astype(f32)
                 * jnp.ones((16, 32), f32))
    w2la = _kron8(W2_lin[:, :32].T)
    w2lb = _kron8(W2_lin[:, 32:].T)
    b2l = _tile8(b2_lin)
    w3m = _kron8(W3_mod.T).astype(jnp.bfloat16)
    b3m = _tile8(b3_mod)
    w3l = _kron8(jnp.pad(W3_lin.T, ((0, 0), (0, 13))))
    b3t = _tile8(jnp.pad(b3_lin, (0, 13)))

    br = pk // 16
    grid = (16,)
    cpt1 = epad // (NC * NS * CH)
    cpt2 = epad // (NS * CH)

    agg16 = _make_agg(rows, 16, cpt1, wv=8, ib=7, edge_split=True)
    agg32 = _make_agg(rows, 32, cpt2, wv=2, ib=7, edge_split=False)

    y1p = pl.pallas_call(
        _premod1_body, grid=grid,
        in_specs=[_row_spec(br, 24), _full_spec((24, 24)),
                  _full_spec((1, 24)), _full_spec((24, 128)),
                  _full_spec((1, 128))],
        out_specs=_row_spec(br, 128),
        out_shape=jax.ShapeDtypeStruct((pk, 128), f32),
    )(x8, w1m, b1m, sp1, ones3)

    m1 = agg16(y1p.reshape(rows, 16), y1p.reshape(rows, 16), srcp, dstp)
    m1p = m1.reshape(2 * pk, 128)

    y2lo, y2hi, rinv = pl.pallas_call(
        _post1_body, grid=grid,
        in_specs=[_row_spec(br, 128), _row_spec_off(br, 128, pk // br),
                  _row_spec(br, 128), _full_spec((128, 128)),
                  _full_spec((128, 256)), _full_spec((128, 256)),
                  _full_spec((1, 256)), _full_spec((1, 256)),
                  _full_spec((256, 256)), _full_spec((256, 256)),
                  _full_spec((256, 256)), _full_spec((256, 256)),
                  _full_spec((1, 256)), _full_spec((1, 256))],
        out_specs=[_row_spec(br, 256), _row_spec(br, 256),
                   _row_spec(br, 128)],
        out_shape=[jax.ShapeDtypeStruct((pk, 256), f32),
                   jax.ShapeDtypeStruct((pk, 256), f32),
                   jax.ShapeDtypeStruct((pk, 128), f32)],
    )(m1p, m1p, y1p, s16, w1la, w1lb, b1la, b1lb,
      w2maa, w2mab, w2mba, w2mbb, b2ma, b2mb)

    m2 = agg32(y2lo.reshape(rows, 32), y2hi.reshape(rows, 32), srcp, dstp)
    m2p = m2.reshape(2 * pk, 256)

    y3p = pl.pallas_call(
        _post2_body, grid=grid,
        in_specs=[_row_spec(br, 256), _row_spec_off(br, 256, pk // br),
                  _row_spec(br, 256), _row_spec(br, 256), _row_spec(br, 128),
                  _full_spec((128, 256)),
                  _full_spec((256, 1024)), _full_spec((256, 1024)),
                  _full_spec((1, 1024)), _full_spec((1024, 1024)),
                  _full_spec((1, 1024)), _full_spec((1024, 128))],
        out_specs=_row_spec(br, 128),
        out_shape=jax.ShapeDtypeStruct((pk, 128), f32),
    )(m2p, m2p, y2lo, y2hi, rinv, r32, w2la, w2lb, b2l, w3m, b3m, w3l)

    m3 = agg16(y3p.reshape(rows, 16), y3p.reshape(rows, 16), srcp, dstp)
    m3p = m3.reshape(2 * pk, 128)

    outp = pl.pallas_call(
        _post3_body, grid=grid,
        in_specs=[_row_spec(br, 128), _row_spec_off(br, 128, pk // br),
                  _row_spec(br, 128), _row_spec(br, 128),
                  _full_spec((1, 128))],
        out_specs=_row_spec(br, 128),
        out_shape=jax.ShapeDtypeStruct((pk, 128), f32),
    )(m3p, m3p, y3p, rinv, b3t)

    return outp.reshape(rows, 16)[:n, :3]


# speedup vs baseline: 37.1827x; 1.0076x over previous
import functools

import jax
import jax.numpy as jnp
from jax import lax
from jax.experimental import pallas as pl
from jax.experimental.pallas import tpu as pltpu
from jax.experimental.pallas import tpu_sc as plsc

NC = 2
NS = 16
CH = 128


def _round_up(a, m):
    return (a + m - 1) // m * m


def _make_agg(rows, d, chunks_per_tile, wv, ib, edge_split):
    rpt = rows // NS
    zb = 56
    n_zc = rpt // zb
    nw = chunks_per_tile // wv
    nb = nw // ib
    assert nw % ib == 0 and rpt % zb == 0 and chunks_per_tile % wv == 0

    mesh = plsc.VectorSubcoreMesh(core_axis_name="c", subcore_axis_name="s")

    @functools.partial(
        pl.kernel,
        out_type=jax.ShapeDtypeStruct((2 * rows, d), jnp.float32),
        mesh=mesh,
        compiler_params=pltpu.CompilerParams(use_tc_tiling_on_sc=False),
        scratch_types=[
            pltpu.VMEM((2, ib * wv, CH), jnp.int32),
            pltpu.VMEM((2, ib * wv, CH), jnp.int32),
            pltpu.VMEM((2, wv, CH, d), jnp.float32),
            pltpu.VMEM((zb, d), jnp.float32),
            pltpu.VMEM_SHARED((rows, d), jnp.float32),
            pltpu.SemaphoreType.DMA,
            pltpu.SemaphoreType.DMA,
            pltpu.SemaphoreType.DMA,
        ],
    )
    def agg(ylo_hbm, yhi_hbm, src_hbm, dst_hbm, out_hbm, sidx, didx, rv,
            zbuf, acc, gsem, ssem, isem):
        c = lax.axis_index("c")
        s = lax.axis_index("s")
        wid = s * NC + c
        tile_base = s * rpt

        def zrow(i, _):
            for k in range(d // 16):
                zbuf[i, 16 * k:16 * (k + 1)] = jnp.zeros((16,), jnp.float32)
            return 0
        lax.fori_loop(0, zb, zrow, 0)

        zr = 14
        for r in range(n_zc // zr):
            def zacc(k, _):
                pltpu.async_copy(zbuf,
                                 acc.at[pl.ds(tile_base + k * zb, zb)], isem)
                return 0
            lax.fori_loop(r * zr, (r + 1) * zr, zacc, 0)

            def zwait(k, _):
                pltpu.make_async_copy(zbuf, acc.at[pl.ds(tile_base, zb)],
                                      isem).wait()
                return 0
            lax.fori_loop(0, zr, zwait, 0)
        plsc.subcore_barrier()

        chunk0 = (wid if edge_split else s) * chunks_per_tile

        def idx_load(b, slot):
            r0 = chunk0 + b * ib * wv
            pltpu.async_copy(src_hbm.at[pl.ds(r0, ib * wv)], sidx.at[slot],
                             isem)
            pltpu.async_copy(dst_hbm.at[pl.ds(r0, ib * wv)], didx.at[slot],
                             isem)

        def wait_idx():
            for _ in range(2):
                pltpu.make_async_copy(src_hbm.at[pl.ds(0, ib * wv)],
                                      sidx.at[0], isem).wait()

        def body(y_hbm):
            def fire_g(g, p, q):
                w = lax.rem(g, ib) * wv
                for j in range(wv):
                    pltpu.async_copy(y_hbm.at[sidx.at[q, w + j]],
                                     rv.at[p, j], gsem)

            def wait_g():
                for j in range(wv):
                    pltpu.make_async_copy(y_hbm.at[pl.ds(0, CH)],
                                          rv.at[0, j], gsem).wait()

            def fire_s(g, p, q):
                w = lax.rem(g, ib) * wv
                for j in range(wv):
                    pltpu.async_copy(rv.at[p, j], acc.at[didx.at[q, w + j]],
                                     ssem, add=True)

            def wait_s():
                for j in range(wv):
                    pltpu.make_async_copy(y_hbm.at[pl.ds(0, CH)],
                                          rv.at[0, j], ssem).wait()

            idx_load(0, 0)
            wait_idx()
            fire_g(0, 0, 0)

            def step(g, _):
                p = lax.rem(g, 2)
                b = g // ib
                q = lax.rem(b, 2)
                wait_g()
                @pl.when(g >= 2)
                def _():
                    wait_s()
                fire_s(g - 1, 1 - p, lax.rem((g - 1) // ib, 2))
                @pl.when(lax.rem(g, ib) == 0)
                def _():
                    wait_idx()
                @pl.when(lax.rem(g, ib) == 1)
                def _():
                    @pl.when(b + 1 < nb)
                    def _():
                        idx_load(b + 1, lax.rem(b + 1, 2))
                fire_g(g, p, q)
                return 0
            lax.fori_loop(1, nw, step, 0)

            wait_g()
            if nw >= 2:
                wait_s()
            fire_s(nw - 1, lax.rem(nw - 1, 2), lax.rem((nw - 1) // ib, 2))
            wait_s()

        if edge_split:
            body(ylo_hbm)
        else:
            @pl.when(c == 0)
            def _():
                body(ylo_hbm)

            @pl.when(c == 1)
            def _():
                body(yhi_hbm)

        plsc.subcore_barrier()
        pltpu.sync_copy(acc.at[pl.ds(tile_base, rpt)],
                        out_hbm.at[pl.ds(c * rows + tile_base, rpt)])

    return agg


def _row_spec(br, d):
    return pl.BlockSpec((br, d), lambda i: (i, 0))


def _row_spec_off(br, d, off_blocks):
    return pl.BlockSpec((br, d), lambda i: (i + off_blocks, 0))


def _full_spec(shape):
    n = len(shape)
    return pl.BlockSpec(shape, lambda i: (0,) * n)


def _premod1_body(x_ref, wm_ref, bm_ref, sp_ref, ones_ref, y_ref):
    xb = x_ref[...]
    m = jax.nn.sigmoid(
        jnp.dot(xb, wm_ref[...], preferred_element_type=jnp.float32)
        + bm_ref[...])
    xm = xb * m
    y_ref[...] = (jnp.dot(xm, sp_ref[...], preferred_element_type=jnp.float32)
                  + ones_ref[...])


def _post1_body(ma_ref, mb_ref, y1_ref, s16_ref, wla_ref, wlb_ref,
                bla_ref, blb_ref, wmaa_ref, wmab_ref, wmba_ref, wmbb_ref,
                bma_ref, bmb_ref, ylo_ref, yhi_ref, rinv_ref):
    msg = ma_ref[...] + mb_ref[...] + y1_ref[...]
    v = jnp.dot(msg, s16_ref[...], preferred_element_type=jnp.float32)
    rinv = 1.0 / v
    aggr = msg * rinv
    h1lo = jax.nn.relu(
        jnp.dot(aggr, wla_ref[...], preferred_element_type=jnp.float32)
        + bla_ref[...])
    h1hi = jax.nn.relu(
        jnp.dot(aggr, wlb_ref[...], preferred_element_type=jnp.float32)
        + blb_ref[...])
    m2lo = jax.nn.sigmoid(
        jnp.dot(h1lo, wmaa_ref[...], preferred_element_type=jnp.float32)
        + jnp.dot(h1hi, wmba_ref[...], preferred_element_type=jnp.float32)
        + bma_ref[...])
    m2hi = jax.nn.sigmoid(
        jnp.dot(h1lo, wmab_ref[...], preferred_element_type=jnp.float32)
        + jnp.dot(h1hi, wmbb_ref[...], preferred_element_type=jnp.float32)
        + bmb_ref[...])
    ylo_ref[...] = h1lo * m2lo
    yhi_ref[...] = h1hi * m2hi
    rinv_ref[...] = rinv


def _post2_body(ma_ref, mb_ref, ylo_ref, yhi_ref, rinv_ref, r32_ref,
                wla_ref, wlb_ref, bl_ref, wm_ref, bm_ref, wl3_ref, y3_ref):
    rs32 = jnp.dot(rinv_ref[...], r32_ref[...],
                   preferred_element_type=jnp.float32)
    alo = (ma_ref[...] + ylo_ref[...]) * rs32
    ahi = (mb_ref[...] + yhi_ref[...]) * rs32
    h2 = jax.nn.relu(
        jnp.dot(alo, wla_ref[...], preferred_element_type=jnp.float32)
        + jnp.dot(ahi, wlb_ref[...], preferred_element_type=jnp.float32)
        + bl_ref[...])
    m3 = jax.nn.sigmoid(
        jnp.dot(h2.astype(jnp.bfloat16), wm_ref[...],
                preferred_element_type=jnp.float32)
        + bm_ref[...])
    xm3 = h2 * m3
    y3_ref[...] = jnp.dot(xm3, wl3_ref[...],
                          preferred_element_type=jnp.float32)


def _post3_body(ma_ref, mb_ref, y3_ref, rinv_ref, bl_ref, o_ref):
    msg = ma_ref[...] + mb_ref[...] + y3_ref[...]
    o_ref[...] = msg * rinv_ref[...] + bl_ref[...]


def _kron8(w):
    return jnp.kron(jnp.eye(8, dtype=jnp.float32), w)


def _tile8(b):
    return jnp.tile(b, 8).reshape(1, -1)


def kernel(x, edge_index, W1_lin, b1_lin, W1_mod, b1_mod,
           W2_lin, b2_lin, W2_mod, b2_mod,
           W3_lin, b3_lin, W3_mod, b3_mod):
    n = x.shape[0]
    e = edge_index.shape[1]
    rows = _round_up(n + 1, 256)
    pk = rows // 8
    epad = _round_up(e, NC * NS * CH * 8)
    ndum = rows - n

    x8 = jnp.pad(x, ((0, rows - n), (0, 0))).reshape(pk, 24)
    pad_idx = (n + (jnp.arange(epad - e, dtype=jnp.int32) % ndum))
    srcp = jnp.concatenate([edge_index[0], pad_idx]).reshape(epad // CH, CH)
    dstp = jnp.concatenate([edge_index[1], pad_idx]).reshape(epad // CH, CH)

    f32 = jnp.float32
    w1m = _kron8(W1_mod.T)
    b1m = _tile8(b1_mod)
    sp1 = _kron8(jnp.pad(jnp.eye(3, dtype=f32), ((0, 0), (0, 13))))
    lane = jnp.arange(128)
    ones3 = jnp.where(lane % 16 == 3, 1.0, 0.0).astype(f32).reshape(1, 128)
    s16 = _kron8((jnp.arange(16)[:, None] == 3).astype(f32)
                 * jnp.ones((16, 16), f32))
    w1la = _kron8(jnp.pad(W1_lin[:32].T, ((0, 13), (0, 0))))
    w1lb = _kron8(jnp.pad(W1_lin[32:].T, ((0, 13), (0, 0))))
    b1la = _tile8(b1_lin[:32])
    b1lb = _tile8(b1_lin[32:])
    w2maa = _kron8(W2_mod[:32, :32].T)
    w2mab = _kron8(W2_mod[32:, :32].T)
    w2mba = _kron8(W2_mod[:32, 32:].T)
    w2mbb = _kron8(W2_mod[32:, 32:].T)
    b2ma = _tile8(b2_mod[:32])
    b2mb = _tile8(b2_mod[32:])
    r32 = _kron8((jnp.arange(16)[:, None] == 0).astype(f32)
                 * jnp.ones((16, 32), f32))
    w2la = _kron8(W2_lin[:, :32].T)
    w2lb = _kron8(W2_lin[:, 32:].T)
    b2l = _tile8(b2_lin)
    w3m = _kron8(W3_mod.T).astype(jnp.bfloat16)
    b3m = _tile8(b3_mod)
    w3l = _kron8(jnp.pad(W3_lin.T, ((0, 0), (0, 13))))
    b3t = _tile8(jnp.pad(b3_lin, (0, 13)))

    br = pk // 16
    grid = (16,)
    cpt1 = epad // (NC * NS * CH)
    cpt2 = epad // (NS * CH)

    agg16 = _make_agg(rows, 16, cpt1, wv=8, ib=7, edge_split=True)
    agg32 = _make_agg(rows, 32, cpt2, wv=2, ib=7, edge_split=False)

    y1p = pl.pallas_call(
        _premod1_body, grid=grid,
        in_specs=[_row_spec(br, 24), _full_spec((24, 24)),
                  _full_spec((1, 24)), _full_spec((24, 128)),
                  _full_spec((1, 128))],
        out_specs=_row_spec(br, 128),
        out_shape=jax.ShapeDtypeStruct((pk, 128), f32),
    )(x8, w1m, b1m, sp1, ones3)

    m1 = agg16(y1p.reshape(rows, 16), y1p.reshape(rows, 16), srcp, dstp)
    m1p = m1.reshape(2 * pk, 128)

    y2lo, y2hi, rinv = pl.pallas_call(
        _post1_body, grid=grid,
        in_specs=[_row_spec(br, 128), _row_spec_off(br, 128, pk // br),
                  _row_spec(br, 128), _full_spec((128, 128)),
                  _full_spec((128, 256)), _full_spec((128, 256)),
                  _full_spec((1, 256)), _full_spec((1, 256)),
                  _full_spec((256, 256)), _full_spec((256, 256)),
                  _full_spec((256, 256)), _full_spec((256, 256)),
                  _full_spec((1, 256)), _full_spec((1, 256))],
        out_specs=[_row_spec(br, 256), _row_spec(br, 256),
                   _row_spec(br, 128)],
        out_shape=[jax.ShapeDtypeStruct((pk, 256), f32),
                   jax.ShapeDtypeStruct((pk, 256), f32),
                   jax.ShapeDtypeStruct((pk, 128), f32)],
    )(m1p, m1p, y1p, s16, w1la, w1lb, b1la, b1lb,
      w2maa, w2mab, w2mba, w2mbb, b2ma, b2mb)

    m2 = agg32(y2lo.reshape(rows, 32), y2hi.reshape(rows, 32), srcp, dstp)
    m2p = m2.reshape(2 * pk, 256)

    y3p = pl.pallas_call(
        _post2_body, grid=grid,
        in_specs=[_row_spec(br, 256), _row_spec_off(br, 256, pk // br),
                  _row_spec(br, 256), _row_spec(br, 256), _row_spec(br, 128),
                  _full_spec((128, 256)),
                  _full_spec((256, 1024)), _full_spec((256, 1024)),
                  _full_spec((1, 1024)), _full_spec((1024, 1024)),
                  _full_spec((1, 1024)), _full_spec((1024, 128))],
        out_specs=_row_spec(br, 128),
        out_shape=jax.ShapeDtypeStruct((pk, 128), f32),
    )(m2p, m2p, y2lo, y2hi, rinv, r32, w2la, w2lb, b2l, w3m, b3m, w3l)

    m3 = agg16(y3p.reshape(rows, 16), y3p.reshape(rows, 16), srcp, dstp)
    m3p = m3.reshape(2 * pk, 128)

    outp = pl.pallas_call(
        _post3_body, grid=grid,
        in_specs=[_row_spec(br, 128), _row_spec_off(br, 128, pk // br),
                  _row_spec(br, 128), _row_spec(br, 128),
                  _full_spec((1, 128))],
        out_specs=_row_spec(br, 128),
        out_shape=jax.ShapeDtypeStruct((pk, 128), f32),
    )(m3p, m3p, y3p, rinv, b3t)

    return outp.reshape(rows, 16)[:n, :3]


# speedup vs baseline: 37.4191x; 1.0064x over previous
import functools

import jax
import jax.numpy as jnp
from jax import lax
from jax.experimental import pallas as pl
from jax.experimental.pallas import tpu as pltpu
from jax.experimental.pallas import tpu_sc as plsc

NC = 2
NS = 16
CH = 128


def _round_up(a, m):
    return (a + m - 1) // m * m


def _make_agg(rows, d, chunks_per_tile, wv, ib, edge_split):
    rpt = rows // NS
    zb = 56
    n_zc = rpt // zb
    nw = chunks_per_tile // wv
    nb = nw // ib
    assert nw % ib == 0 and rpt % zb == 0 and chunks_per_tile % wv == 0

    mesh = plsc.VectorSubcoreMesh(core_axis_name="c", subcore_axis_name="s")

    @functools.partial(
        pl.kernel,
        out_type=jax.ShapeDtypeStruct((2 * rows, d), jnp.float32),
        mesh=mesh,
        compiler_params=pltpu.CompilerParams(use_tc_tiling_on_sc=False),
        scratch_types=[
            pltpu.VMEM((2, ib * wv, CH), jnp.int32),
            pltpu.VMEM((2, ib * wv, CH), jnp.int32),
            pltpu.VMEM((2, wv, CH, d), jnp.float32),
            pltpu.VMEM((zb, d), jnp.float32),
            pltpu.VMEM_SHARED((rows, d), jnp.float32),
            pltpu.SemaphoreType.DMA,
            pltpu.SemaphoreType.DMA,
            pltpu.SemaphoreType.DMA,
        ],
    )
    def agg(ylo_hbm, yhi_hbm, src_hbm, dst_hbm, out_hbm, sidx, didx, rv,
            zbuf, acc, gsem, ssem, isem):
        c = lax.axis_index("c")
        s = lax.axis_index("s")
        wid = s * NC + c
        tile_base = s * rpt

        def zrow(i, _):
            for k in range(d // 16):
                zbuf[i, 16 * k:16 * (k + 1)] = jnp.zeros((16,), jnp.float32)
            return 0
        lax.fori_loop(0, zb, zrow, 0)

        zr = 14
        for r in range(n_zc // zr):
            def zacc(k, _):
                pltpu.async_copy(zbuf,
                                 acc.at[pl.ds(tile_base + k * zb, zb)], isem)
                return 0
            lax.fori_loop(r * zr, (r + 1) * zr, zacc, 0)

            def zwait(k, _):
                pltpu.make_async_copy(zbuf, acc.at[pl.ds(tile_base, zb)],
                                      isem).wait()
                return 0
            lax.fori_loop(0, zr, zwait, 0)
        plsc.subcore_barrier()

        chunk0 = (wid if edge_split else s) * chunks_per_tile

        def idx_load(b, slot):
            r0 = chunk0 + b * ib * wv
            pltpu.async_copy(src_hbm.at[pl.ds(r0, ib * wv)], sidx.at[slot],
                             isem)
            pltpu.async_copy(dst_hbm.at[pl.ds(r0, ib * wv)], didx.at[slot],
                             isem)

        def wait_idx():
            for _ in range(2):
                pltpu.make_async_copy(src_hbm.at[pl.ds(0, ib * wv)],
                                      sidx.at[0], isem).wait()

        def body(y_hbm):
            def fire_g(g, p, q):
                w = lax.rem(g, ib) * wv
                for j in range(wv):
                    pltpu.async_copy(y_hbm.at[sidx.at[q, w + j]],
                                     rv.at[p, j], gsem)

            def wait_g():
                for j in range(wv):
                    pltpu.make_async_copy(y_hbm.at[pl.ds(0, CH)],
                                          rv.at[0, j], gsem).wait()

            def fire_s(g, p, q):
                w = lax.rem(g, ib) * wv
                for j in range(wv):
                    pltpu.async_copy(rv.at[p, j], acc.at[didx.at[q, w + j]],
                                     ssem, add=True)

            def wait_s():
                for j in range(wv):
                    pltpu.make_async_copy(y_hbm.at[pl.ds(0, CH)],
                                          rv.at[0, j], ssem).wait()

            idx_load(0, 0)
            wait_idx()
            fire_g(0, 0, 0)

            def step(g, _):
                p = lax.rem(g, 2)
                b = g // ib
                q = lax.rem(b, 2)
                wait_g()
                @pl.when(g >= 2)
                def _():
                    wait_s()
                fire_s(g - 1, 1 - p, lax.rem((g - 1) // ib, 2))
                @pl.when(lax.rem(g, ib) == 0)
                def _():
                    wait_idx()
                @pl.when(lax.rem(g, ib) == 1)
                def _():
                    @pl.when(b + 1 < nb)
                    def _():
                        idx_load(b + 1, lax.rem(b + 1, 2))
                fire_g(g, p, q)
                return 0
            lax.fori_loop(1, nw, step, 0)

            wait_g()
            if nw >= 2:
                wait_s()
            fire_s(nw - 1, lax.rem(nw - 1, 2), lax.rem((nw - 1) // ib, 2))
            wait_s()

        if edge_split:
            body(ylo_hbm)
        else:
            @pl.when(c == 0)
            def _():
                body(ylo_hbm)

            @pl.when(c == 1)
            def _():
                body(yhi_hbm)

        plsc.subcore_barrier()
        pltpu.sync_copy(acc.at[pl.ds(tile_base, rpt)],
                        out_hbm.at[pl.ds(c * rows + tile_base, rpt)])

    return agg


def _row_spec(br, d):
    return pl.BlockSpec((br, d), lambda i: (i, 0))


def _row_spec_off(br, d, off_blocks):
    return pl.BlockSpec((br, d), lambda i: (i + off_blocks, 0))


def _full_spec(shape):
    n = len(shape)
    return pl.BlockSpec(shape, lambda i: (0,) * n)


def _premod1_body(x_ref, wm_ref, bm_ref, sp_ref, ones_ref, y_ref):
    xb = x_ref[...]
    m = jax.nn.sigmoid(
        jnp.dot(xb, wm_ref[...], preferred_element_type=jnp.float32)
        + bm_ref[...])
    xm = xb * m
    y_ref[...] = (jnp.dot(xm, sp_ref[...], preferred_element_type=jnp.float32)
                  + ones_ref[...])


def _post1_body(ma_ref, mb_ref, y1_ref, s16_ref, wla_ref, wlb_ref,
                bla_ref, blb_ref, wmaa_ref, wmab_ref, wmba_ref, wmbb_ref,
                bma_ref, bmb_ref, ylo_ref, yhi_ref, rinv_ref):
    msg = ma_ref[...] + mb_ref[...] + y1_ref[...]
    v = jnp.dot(msg, s16_ref[...], preferred_element_type=jnp.float32)
    rinv = 1.0 / v
    aggr = msg * rinv
    h1lo = jax.nn.relu(
        jnp.dot(aggr, wla_ref[...], preferred_element_type=jnp.float32)
        + bla_ref[...])
    h1hi = jax.nn.relu(
        jnp.dot(aggr, wlb_ref[...], preferred_element_type=jnp.float32)
        + blb_ref[...])
    m2lo = jax.nn.sigmoid(
        jnp.dot(h1lo, wmaa_ref[...], preferred_element_type=jnp.float32)
        + jnp.dot(h1hi, wmba_ref[...], preferred_element_type=jnp.float32)
        + bma_ref[...])
    m2hi = jax.nn.sigmoid(
        jnp.dot(h1lo, wmab_ref[...], preferred_element_type=jnp.float32)
        + jnp.dot(h1hi, wmbb_ref[...], preferred_element_type=jnp.float32)
        + bmb_ref[...])
    ylo_ref[...] = h1lo * m2lo
    yhi_ref[...] = h1hi * m2hi
    rinv_ref[...] = rinv


def _post2_body(ma_ref, mb_ref, ylo_ref, yhi_ref, rinv_ref, r32_ref,
                wla_ref, wlb_ref, bl_ref, wm_ref, bm_ref, wl3_ref, y3_ref):
    rs32 = jnp.dot(rinv_ref[...], r32_ref[...],
                   preferred_element_type=jnp.float32)
    alo = (ma_ref[...] + ylo_ref[...]) * rs32
    ahi = (mb_ref[...] + yhi_ref[...]) * rs32
    h2 = jax.nn.relu(
        jnp.dot(alo, wla_ref[...], preferred_element_type=jnp.float32)
        + jnp.dot(ahi, wlb_ref[...], preferred_element_type=jnp.float32)
        + bl_ref[...])
    m3 = jax.nn.sigmoid(
        jnp.dot(h2.astype(jnp.bfloat16), wm_ref[...],
                preferred_element_type=jnp.float32)
        + bm_ref[...])
    xm3 = h2 * m3
    y3_ref[...] = jnp.dot(xm3, wl3_ref[...],
                          preferred_element_type=jnp.float32)


def _post3_body(ma_ref, mb_ref, y3_ref, rinv_ref, bl_ref, o_ref):
    msg = ma_ref[...] + mb_ref[...] + y3_ref[...]
    o_ref[...] = msg * rinv_ref[...] + bl_ref[...]


def _kron8(w):
    return jnp.kron(jnp.eye(8, dtype=jnp.float32), w)


def _tile8(b):
    return jnp.tile(b, 8).reshape(1, -1)


def kernel(x, edge_index, W1_lin, b1_lin, W1_mod, b1_mod,
           W2_lin, b2_lin, W2_mod, b2_mod,
           W3_lin, b3_lin, W3_mod, b3_mod):
    n = x.shape[0]
    e = edge_index.shape[1]
    rows = _round_up(n + 1, 256)
    pk = rows // 8
    epad = _round_up(e, NC * NS * CH * 8)
    ndum = rows - n

    x8 = jnp.pad(x, ((0, rows - n), (0, 0))).reshape(pk, 24)

    ecols = 16384

    def _eprep_body(ei_ref, s_ref, d_ref):
        i = pl.program_id(0)
        pos = i * ecols + lax.broadcasted_iota(jnp.int32, (1, ecols), 1)
        dummy = n + lax.rem(pos, ndum)
        ok = pos < e
        blk = ei_ref[...]
        s_ref[...] = jnp.reshape(jnp.where(ok, blk[0:1, :], dummy),
                                 (ecols // CH, CH))
        d_ref[...] = jnp.reshape(jnp.where(ok, blk[1:2, :], dummy),
                                 (ecols // CH, CH))

    srcp, dstp = pl.pallas_call(
        _eprep_body, grid=(epad // ecols,),
        in_specs=[pl.BlockSpec((2, ecols), lambda i: (0, i))],
        out_specs=[pl.BlockSpec((ecols // CH, CH), lambda i: (i, 0)),
                   pl.BlockSpec((ecols // CH, CH), lambda i: (i, 0))],
        out_shape=[jax.ShapeDtypeStruct((epad // CH, CH), jnp.int32),
                   jax.ShapeDtypeStruct((epad // CH, CH), jnp.int32)],
    )(edge_index)

    f32 = jnp.float32
    w1m = _kron8(W1_mod.T)
    b1m = _tile8(b1_mod)
    sp1 = _kron8(jnp.pad(jnp.eye(3, dtype=f32), ((0, 0), (0, 13))))
    lane = jnp.arange(128)
    ones3 = jnp.where(lane % 16 == 3, 1.0, 0.0).astype(f32).reshape(1, 128)
    s16 = _kron8((jnp.arange(16)[:, None] == 3).astype(f32)
                 * jnp.ones((16, 16), f32))
    w1la = _kron8(jnp.pad(W1_lin[:32].T, ((0, 13), (0, 0))))
    w1lb = _kron8(jnp.pad(W1_lin[32:].T, ((0, 13), (0, 0))))
    b1la = _tile8(b1_lin[:32])
    b1lb = _tile8(b1_lin[32:])
    w2maa = _kron8(W2_mod[:32, :32].T)
    w2mab = _kron8(W2_mod[32:, :32].T)
    w2mba = _kron8(W2_mod[:32, 32:].T)
    w2mbb = _kron8(W2_mod[32:, 32:].T)
    b2ma = _tile8(b2_mod[:32])
    b2mb = _tile8(b2_mod[32:])
    r32 = _kron8((jnp.arange(16)[:, None] == 0).astype(f32)
                 * jnp.ones((16, 32), f32))
    w2la = _kron8(W2_lin[:, :32].T)
    w2lb = _kron8(W2_lin[:, 32:].T)
    b2l = _tile8(b2_lin)
    w3m = _kron8(W3_mod.T).astype(jnp.bfloat16)
    b3m = _tile8(b3_mod)
    w3l = _kron8(jnp.pad(W3_lin.T, ((0, 0), (0, 13))))
    b3t = _tile8(jnp.pad(b3_lin, (0, 13)))

    br = pk // 16
    grid = (16,)
    cpt1 = epad // (NC * NS * CH)
    cpt2 = epad // (NS * CH)

    agg16 = _make_agg(rows, 16, cpt1, wv=8, ib=7, edge_split=True)
    agg32 = _make_agg(rows, 32, cpt2, wv=2, ib=7, edge_split=False)

    y1p = pl.pallas_call(
        _premod1_body, grid=grid,
        in_specs=[_row_spec(br, 24), _full_spec((24, 24)),
                  _full_spec((1, 24)), _full_spec((24, 128)),
                  _full_spec((1, 128))],
        out_specs=_row_spec(br, 128),
        out_shape=jax.ShapeDtypeStruct((pk, 128), f32),
    )(x8, w1m, b1m, sp1, ones3)

    m1p = agg16(y1p.reshape(rows, 16), y1p.reshape(rows, 16), srcp,
                dstp).reshape(2 * pk, 128)

    y2lo, y2hi, rinv = pl.pallas_call(
        _post1_body, grid=grid,
        in_specs=[_row_spec(br, 128), _row_spec_off(br, 128, pk // br),
                  _row_spec(br, 128), _full_spec((128, 128)),
                  _full_spec((128, 256)), _full_spec((128, 256)),
                  _full_spec((1, 256)), _full_spec((1, 256)),
                  _full_spec((256, 256)), _full_spec((256, 256)),
                  _full_spec((256, 256)), _full_spec((256, 256)),
                  _full_spec((1, 256)), _full_spec((1, 256))],
        out_specs=[_row_spec(br, 256), _row_spec(br, 256),
                   _row_spec(br, 128)],
        out_shape=[jax.ShapeDtypeStruct((pk, 256), f32),
                   jax.ShapeDtypeStruct((pk, 256), f32),
                   jax.ShapeDtypeStruct((pk, 128), f32)],
    )(m1p, m1p, y1p, s16, w1la, w1lb, b1la, b1lb,
      w2maa, w2mab, w2mba, w2mbb, b2ma, b2mb)

    m2p = agg32(y2lo.reshape(rows, 32), y2hi.reshape(rows, 32), srcp,
                dstp).reshape(2 * pk, 256)

    y3p = pl.pallas_call(
        _post2_body, grid=grid,
        in_specs=[_row_spec(br, 256), _row_spec_off(br, 256, pk // br),
                  _row_spec(br, 256), _row_spec(br, 256), _row_spec(br, 128),
                  _full_spec((128, 256)),
                  _full_spec((256, 1024)), _full_spec((256, 1024)),
                  _full_spec((1, 1024)), _full_spec((1024, 1024)),
                  _full_spec((1, 1024)), _full_spec((1024, 128))],
        out_specs=_row_spec(br, 128),
        out_shape=jax.ShapeDtypeStruct((pk, 128), f32),
    )(m2p, m2p, y2lo, y2hi, rinv, r32, w2la, w2lb, b2l, w3m, b3m, w3l)

    m3p = agg16(y3p.reshape(rows, 16), y3p.reshape(rows, 16), srcp,
                dstp).reshape(2 * pk, 128)

    outp = pl.pallas_call(
        _post3_body, grid=grid,
        in_specs=[_row_spec(br, 128), _row_spec_off(br, 128, pk // br),
                  _row_spec(br, 128), _row_spec(br, 128),
                  _full_spec((1, 128))],
        out_specs=_row_spec(br, 128),
        out_shape=jax.ShapeDtypeStruct((pk, 128), f32),
    )(m3p, m3p, y3p, rinv, b3t)

    return outp.reshape(rows, 16)[:n, :3]
